# Initial kernel scaffold; baseline (speedup 1.0000x reference)
import functools

import jax
import jax.numpy as jnp
from jax import lax
from jax.experimental import pallas as pl
from jax.experimental.pallas import tpu as pltpu
from jax.experimental.pallas import tpu_sc as plsc

C = 4
N = 10000
E = 160000
D = 256
HD = 128
NT = 16
EPT = E // NT
K = 80
NCH = EPT // K
NPT = N // NT

_mesh = plsc.VectorSubcoreMesh(core_axis_name="c", subcore_axis_name="s")


@functools.partial(
    pl.kernel,
    mesh=_mesh,
    out_type=jax.ShapeDtypeStruct((C, 2, N), jnp.float32),
    scratch_types=[
        pltpu.VMEM((N,), jnp.float32),
        pltpu.VMEM((160,), jnp.int32),
    ],
)
def _deg_sc(edges_ref, deg_ref, hist, idxb):
    c = lax.axis_index("c")
    s = lax.axis_index("s")
    job = c * 4 + s
    for jid in range(8):

        @pl.when(jnp.logical_and(s < 4, job == jid))
        def _():
            cut, which = jid // 2, jid % 2

            def zb(i, _):
                hist[pl.ds(i * 16, 16)] = jnp.zeros((16,), jnp.float32)
                return 0

            lax.fori_loop(0, N // 16, zb, 0)
            ones = jnp.ones((16,), jnp.float32)

            def chunk(i, _):
                pltpu.sync_copy(edges_ref.at[cut, which, pl.ds(i * 160, 160)], idxb)
                for k in range(10):
                    v = idxb[pl.ds(k * 16, 16)]
                    plsc.addupdate_scatter(hist, [v], ones)
                return 0

            lax.fori_loop(0, E // 160, chunk, 0)
            pltpu.sync_copy(hist, deg_ref.at[cut, which])


@functools.partial(
    pl.kernel,
    mesh=_mesh,
    out_type=jax.ShapeDtypeStruct((C, 2, N, HD), jnp.float32),
    scratch_types=[
        pltpu.VMEM_SHARED((N, HD), jnp.float32),
        pltpu.VMEM((K, HD), jnp.float32),
        pltpu.VMEM((K,), jnp.int32),
        pltpu.VMEM((1, K), jnp.int32),
        pltpu.VMEM((125, HD), jnp.float32),
        pltpu.SemaphoreType.DMA,
    ],
)
def _agg_sc(hn_ref, edges_ref, out_ref, acc, rows, idx_s, idx_d, zbuf, sem):
    c = lax.axis_index("c")
    s = lax.axis_index("s")

    def zb(i, _):
        for k in range(HD // 16):
            zbuf[i, pl.ds(k * 16, 16)] = jnp.zeros((16,), jnp.float32)
        return 0

    lax.fori_loop(0, 125, zb, 0)

    for hl in range(2):

        @pl.when(c == hl)
        def _():
            for cut in range(C):
                base = s * NPT
                for z in range(5):
                    pltpu.sync_copy(zbuf, acc.at[pl.ds(base + z * 125, 125)])
                plsc.subcore_barrier()

                def chunk(i, _):
                    off = s * EPT + i * K
                    pltpu.sync_copy(edges_ref.at[cut, 0, pl.ds(off, K)], idx_s)
                    pltpu.sync_copy(edges_ref.at[cut, 1, pl.ds(off, K)], idx_d.at[0])
                    pltpu.async_copy(hn_ref.at[cut, hl].at[idx_s], rows, sem).wait()
                    pltpu.sync_copy(rows, acc.at[idx_d.at[0]], add=True)
                    return 0

                lax.fori_loop(0, NCH, chunk, 0)
                plsc.subcore_barrier()
                pltpu.sync_copy(
                    acc.at[pl.ds(base, NPT)], out_ref.at[cut, hl, pl.ds(base, NPT)]
                )
                plsc.subcore_barrier()


_BN = 1000


def _scale_body(x_ref, degt_ref, out_ref):
    deg = degt_ref[0]
    ns = lax.rsqrt(jnp.maximum(deg[:, 0:1], 1.0))
    h = x_ref[0] * ns
    out_ref[0, 0] = h[:, :HD]
    out_ref[0, 1] = h[:, HD:]


def _scale_tc(x, degt):
    return pl.pallas_call(
        _scale_body,
        grid=(C, N // _BN),
        in_specs=[
            pl.BlockSpec((1, _BN, D), lambda ci, ni: (ci, ni, 0)),
            pl.BlockSpec((1, _BN, 2), lambda ci, ni: (ci, ni, 0)),
        ],
        out_specs=pl.BlockSpec((1, 2, _BN, HD), lambda ci, ni: (ci, 0, ni, 0)),
        out_shape=jax.ShapeDtypeStruct((C, 2, N, HD), jnp.float32),
    )(x, degt)


def _layer_body(mid, agg_ref, degt_ref, w_ref, b_ref, out_ref):
    a = jnp.concatenate([agg_ref[0, 0], agg_ref[0, 1]], axis=-1)
    deg = degt_ref[0]
    nd = lax.rsqrt(jnp.maximum(deg[:, 1:2], 1.0))
    a = a * nd
    h = (
        jnp.dot(a, w_ref[0], preferred_element_type=jnp.float32,
                precision=lax.Precision.HIGHEST)
        + b_ref[0]
    )
    h = jnp.where(h >= 0.0, h, 0.01 * h)
    if mid:
        ns = lax.rsqrt(jnp.maximum(deg[:, 0:1], 1.0))
        h = h * ns
        out_ref[0, 0] = h[:, :HD]
        out_ref[0, 1] = h[:, HD:]
    else:
        out_ref[0] = h


def _layer_tc(agg, degt, w, b, mid):
    if mid:
        out_spec = pl.BlockSpec((1, 2, _BN, HD), lambda ci, ni: (ci, 0, ni, 0))
        out_shape = jax.ShapeDtypeStruct((C, 2, N, HD), jnp.float32)
    else:
        out_spec = pl.BlockSpec((1, _BN, D), lambda ci, ni: (ci, ni, 0))
        out_shape = jax.ShapeDtypeStruct((C, N, D), jnp.float32)
    return pl.pallas_call(
        functools.partial(_layer_body, mid),
        grid=(C, N // _BN),
        in_specs=[
            pl.BlockSpec((1, 2, _BN, HD), lambda ci, ni: (ci, 0, ni, 0)),
            pl.BlockSpec((1, _BN, 2), lambda ci, ni: (ci, ni, 0)),
            pl.BlockSpec((1, D, D), lambda ci, ni: (ci, 0, 0)),
            pl.BlockSpec((1, D), lambda ci, ni: (ci, 0)),
        ],
        out_specs=out_spec,
        out_shape=out_shape,
    )(agg, degt, w, b)


def _gru_body(feats_ref, wi_ref, wh_ref, bi_ref, bh_ref, out_ref):
    h = jnp.zeros((_BN, D), jnp.float32)
    wi = wi_ref[...]
    wh = wh_ref[...]
    bi = bi_ref[0]
    bh = bh_ref[0]
    dn = (((1,), (1,)), ((), ()))
    for t in range(C):
        xt = feats_ref[t]
        gi = lax.dot_general(xt, wi, dn, preferred_element_type=jnp.float32,
                             precision=lax.Precision.HIGHEST) + bi
        gh = lax.dot_general(h, wh, dn, preferred_element_type=jnp.float32,
                             precision=lax.Precision.HIGHEST) + bh
        r = jax.nn.sigmoid(gi[:, :D] + gh[:, :D])
        z = jax.nn.sigmoid(gi[:, D : 2 * D] + gh[:, D : 2 * D])
        n = jnp.tanh(gi[:, 2 * D :] + r * gh[:, 2 * D :])
        h = (1.0 - z) * n + z * h
    out_ref[...] = h


def _gru_tc(feats, w_ih, w_hh, b_ih, b_hh):
    return pl.pallas_call(
        _gru_body,
        grid=(N // _BN,),
        in_specs=[
            pl.BlockSpec((C, _BN, D), lambda ni: (0, ni, 0)),
            pl.BlockSpec((3 * D, D), lambda ni: (0, 0)),
            pl.BlockSpec((3 * D, D), lambda ni: (0, 0)),
            pl.BlockSpec((1, 3 * D), lambda ni: (0, 0)),
            pl.BlockSpec((1, 3 * D), lambda ni: (0, 0)),
        ],
        out_specs=pl.BlockSpec((_BN, D), lambda ni: (ni, 0)),
        out_shape=jax.ShapeDtypeStruct((N, D), jnp.float32),
    )(feats, w_ih, w_hh, b_ih, b_hh)


def kernel(x, edge_index, W_gnn, b_gnn, W_ih, W_hh, b_ih, b_hh):
    edges = edge_index.astype(jnp.int32)
    deg = _deg_sc(edges)
    degt = jnp.transpose(deg, (0, 2, 1))
    hn0 = _scale_tc(x, degt)
    agg0 = _agg_sc(hn0, edges)
    hn1 = _layer_tc(agg0, degt, W_gnn[:, 0], b_gnn[:, 0], mid=True)
    agg1 = _agg_sc(hn1, edges)
    feats = _layer_tc(agg1, degt, W_gnn[:, 1], b_gnn[:, 1], mid=False)
    return _gru_tc(feats, W_ih, W_hh, b_ih.reshape(1, 3 * D),
                   b_hh.reshape(1, 3 * D))


# speedup vs baseline: 2.0129x; 2.0129x over previous
import functools

import jax
import jax.numpy as jnp
from jax import lax
from jax.experimental import pallas as pl
from jax.experimental.pallas import tpu as pltpu
from jax.experimental.pallas import tpu_sc as plsc

C = 4
N = 10000
E = 160000
D = 256
HD = 128
NT = 16
EPT = E // NT
K = 80
NCH = EPT // K
NPT = N // NT

_mesh = plsc.VectorSubcoreMesh(core_axis_name="c", subcore_axis_name="s")


@functools.partial(
    pl.kernel,
    mesh=_mesh,
    compiler_params=pltpu.CompilerParams(needs_layout_passes=False),
    out_type=jax.ShapeDtypeStruct((C * 2 * N,), jnp.float32),
    scratch_types=[
        pltpu.VMEM((N,), jnp.float32),
        pltpu.VMEM((160,), jnp.int32),
    ],
)
def _deg_sc(srcs_ref, dsts_ref, deg_ref, hist, idxb):
    c = lax.axis_index("c")
    s = lax.axis_index("s")
    job = c * 4 + s
    for jid in range(8):

        @pl.when(jnp.logical_and(s < 4, job == jid))
        def _():
            cut, which = jid // 2, jid % 2
            e_ref = srcs_ref if which == 0 else dsts_ref

            def zb(i, _):
                hist[pl.ds(i * 16, 16)] = jnp.zeros((16,), jnp.float32)
                return 0

            lax.fori_loop(0, N // 16, zb, 0)
            ones = jnp.ones((16,), jnp.float32)

            def chunk(i, _):
                pltpu.sync_copy(e_ref.at[pl.ds(cut * E + i * 160, 160)], idxb)
                for k in range(10):
                    v = idxb[pl.ds(k * 16, 16)]
                    plsc.addupdate_scatter(hist, [v], ones)
                return 0

            lax.fori_loop(0, E // 160, chunk, 0)
            pltpu.sync_copy(hist, deg_ref.at[pl.ds((cut * 2 + which) * N, N)])


@functools.partial(
    pl.kernel,
    mesh=_mesh,
    out_type=jax.ShapeDtypeStruct((C, 2, N, HD), jnp.float32),
    scratch_types=[
        pltpu.VMEM_SHARED((N, HD), jnp.float32),
        pltpu.VMEM((K, HD), jnp.float32),
        pltpu.VMEM((K,), jnp.int32),
        pltpu.VMEM((1, K), jnp.int32),
        pltpu.VMEM((160, HD), jnp.float32),
        pltpu.SemaphoreType.DMA,
    ],
)
def _agg_sc(hn_ref, srcs_ref, dsts_ref, out_ref, acc, rows, idx_s, idx_d, zbuf, sem):
    c = lax.axis_index("c")
    s = lax.axis_index("s")

    def zb(i, _):
        for k in range(HD // 16):
            zbuf[i, pl.ds(k * 16, 16)] = jnp.zeros((16,), jnp.float32)
        return 0

    lax.fori_loop(0, 160, zb, 0)

    for hl in range(2):

        @pl.when(c == hl)
        def _():
            for cut in range(C):
                base = s * 624
                for z in range(4):
                    pltpu.sync_copy(zbuf, acc.at[pl.ds(base + z * 160, 160)])
                plsc.subcore_barrier()

                def chunk(i, _):
                    off = cut * E + s * EPT + i * K
                    pltpu.sync_copy(srcs_ref.at[pl.ds(off, K)], idx_s)
                    pltpu.sync_copy(dsts_ref.at[pl.ds(off, K)], idx_d.at[0])
                    pltpu.async_copy(hn_ref.at[cut, hl].at[idx_s], rows, sem).wait()
                    pltpu.sync_copy(rows, acc.at[idx_d.at[0]], add=True)
                    return 0

                lax.fori_loop(0, NCH, chunk, 0)
                plsc.subcore_barrier()
                pltpu.sync_copy(
                    acc.at[pl.ds(base, 640)], out_ref.at[cut, hl, pl.ds(base, 640)]
                )
                plsc.subcore_barrier()


_BN = 1000


def _scale_body(x_ref, degt_ref, out_ref):
    deg = degt_ref[0]
    ns = lax.rsqrt(jnp.maximum(deg[:, 0:1], 1.0))
    h = x_ref[0] * ns
    out_ref[0, 0] = h[:, :HD]
    out_ref[0, 1] = h[:, HD:]


def _scale_tc(x, degt):
    return pl.pallas_call(
        _scale_body,
        grid=(C, N // _BN),
        in_specs=[
            pl.BlockSpec((1, _BN, D), lambda ci, ni: (ci, ni, 0)),
            pl.BlockSpec((1, _BN, 2), lambda ci, ni: (ci, ni, 0)),
        ],
        out_specs=pl.BlockSpec((1, 2, _BN, HD), lambda ci, ni: (ci, 0, ni, 0)),
        out_shape=jax.ShapeDtypeStruct((C, 2, N, HD), jnp.float32),
    )(x, degt)


def _layer_body(mid, agg_ref, degt_ref, w_ref, b_ref, out_ref):
    a = jnp.concatenate([agg_ref[0, 0], agg_ref[0, 1]], axis=-1)
    deg = degt_ref[0]
    nd = lax.rsqrt(jnp.maximum(deg[:, 1:2], 1.0))
    a = a * nd
    h = (
        jnp.dot(a, w_ref[0], preferred_element_type=jnp.float32,
                precision=lax.Precision.HIGHEST)
        + b_ref[0, 0]
    )
    h = jnp.where(h >= 0.0, h, 0.01 * h)
    if mid:
        ns = lax.rsqrt(jnp.maximum(deg[:, 0:1], 1.0))
        h = h * ns
        out_ref[0, 0] = h[:, :HD]
        out_ref[0, 1] = h[:, HD:]
    else:
        out_ref[0] = h


def _layer_tc(agg, degt, w, b, mid):
    if mid:
        out_spec = pl.BlockSpec((1, 2, _BN, HD), lambda ci, ni: (ci, 0, ni, 0))
        out_shape = jax.ShapeDtypeStruct((C, 2, N, HD), jnp.float32)
    else:
        out_spec = pl.BlockSpec((1, _BN, D), lambda ci, ni: (ci, ni, 0))
        out_shape = jax.ShapeDtypeStruct((C, N, D), jnp.float32)
    return pl.pallas_call(
        functools.partial(_layer_body, mid),
        grid=(C, N // _BN),
        in_specs=[
            pl.BlockSpec((1, 2, _BN, HD), lambda ci, ni: (ci, 0, ni, 0)),
            pl.BlockSpec((1, _BN, 2), lambda ci, ni: (ci, ni, 0)),
            pl.BlockSpec((1, D, D), lambda ci, ni: (ci, 0, 0)),
            pl.BlockSpec((1, 1, D), lambda ci, ni: (ci, 0, 0)),
        ],
        out_specs=out_spec,
        out_shape=out_shape,
    )(agg, degt, w, b.reshape(C, 1, D))


def _gru_body(feats_ref, wi_ref, wh_ref, bi_ref, bh_ref, out_ref):
    h = jnp.zeros((_BN, D), jnp.float32)
    wi = wi_ref[...]
    wh = wh_ref[...]
    bi = bi_ref[0]
    bh = bh_ref[0]
    dn = (((1,), (1,)), ((), ()))
    for t in range(C):
        xt = feats_ref[t]
        gi = lax.dot_general(xt, wi, dn, preferred_element_type=jnp.float32,
                             precision=lax.Precision.HIGHEST) + bi
        gh = lax.dot_general(h, wh, dn, preferred_element_type=jnp.float32,
                             precision=lax.Precision.HIGHEST) + bh
        r = jax.nn.sigmoid(gi[:, :D] + gh[:, :D])
        z = jax.nn.sigmoid(gi[:, D : 2 * D] + gh[:, D : 2 * D])
        n = jnp.tanh(gi[:, 2 * D :] + r * gh[:, 2 * D :])
        h = (1.0 - z) * n + z * h
    out_ref[...] = h


def _gru_tc(feats, w_ih, w_hh, b_ih, b_hh):
    return pl.pallas_call(
        _gru_body,
        grid=(N // _BN,),
        in_specs=[
            pl.BlockSpec((C, _BN, D), lambda ni: (0, ni, 0)),
            pl.BlockSpec((3 * D, D), lambda ni: (0, 0)),
            pl.BlockSpec((3 * D, D), lambda ni: (0, 0)),
            pl.BlockSpec((1, 3 * D), lambda ni: (0, 0)),
            pl.BlockSpec((1, 3 * D), lambda ni: (0, 0)),
        ],
        out_specs=pl.BlockSpec((_BN, D), lambda ni: (ni, 0)),
        out_shape=jax.ShapeDtypeStruct((N, D), jnp.float32),
    )(feats, w_ih, w_hh, b_ih, b_hh)


def kernel(x, edge_index, W_gnn, b_gnn, W_ih, W_hh, b_ih, b_hh):
    edges = edge_index.astype(jnp.int32)
    srcs = edges[:, 0, :].reshape(C * E)
    dsts = edges[:, 1, :].reshape(C * E)
    deg = _deg_sc(srcs, dsts).reshape(C, 2, N)
    degt = jnp.transpose(deg, (0, 2, 1))
    hn0 = _scale_tc(x, degt)
    agg0 = _agg_sc(hn0, srcs, dsts)
    hn1 = _layer_tc(agg0, degt, W_gnn[:, 0], b_gnn[:, 0], mid=True)
    agg1 = _agg_sc(hn1, srcs, dsts)
    feats = _layer_tc(agg1, degt, W_gnn[:, 1], b_gnn[:, 1], mid=False)
    return _gru_tc(feats, W_ih, W_hh, b_ih.reshape(1, 3 * D),
                   b_hh.reshape(1, 3 * D))


# speedup vs baseline: 2.6713x; 1.3271x over previous
import functools

import jax
import jax.numpy as jnp
from jax import lax
from jax.experimental import pallas as pl
from jax.experimental.pallas import tpu as pltpu
from jax.experimental.pallas import tpu_sc as plsc

C = 4
N = 10000
E = 160000
D = 256
HD = 128
NT = 16
EPT = E // NT
K = 128
EPTP = 10240
NCH = EPTP // K
PAD = EPTP - EPT
NPT = N // NT

_mesh = plsc.VectorSubcoreMesh(core_axis_name="c", subcore_axis_name="s")


@functools.partial(
    pl.kernel,
    mesh=_mesh,
    compiler_params=pltpu.CompilerParams(needs_layout_passes=False),
    out_type=jax.ShapeDtypeStruct((4 * C * 2 * N,), jnp.float32),
    scratch_types=[
        pltpu.VMEM((N,), jnp.float32),
        pltpu.VMEM((E // 4,), jnp.int32),
    ],
)
def _deg_sc(srcs4_ref, dsts4_ref, degp_ref, hist, idxb):
    c = lax.axis_index("c")
    s = lax.axis_index("s")
    j = c * 4 + s // 4
    p = s % 4
    EQ = E // 4
    for jid in range(8):

        @pl.when(j == jid)
        def _():
            cut, which = jid // 2, jid % 2
            e_ref = srcs4_ref if which == 0 else dsts4_ref

            def zb(i, _):
                hist[pl.ds(i * 16, 16)] = jnp.zeros((16,), jnp.float32)
                return 0

            lax.fori_loop(0, N // 16, zb, 0)
            pltpu.sync_copy(e_ref.at[cut * 4 + p], idxb)
            ones = jnp.ones((16,), jnp.float32)

            def chunk(i, _):
                for k in range(10):
                    v = idxb[pl.ds(i * 160 + k * 16, 16)]
                    plsc.addupdate_scatter(hist, [v], ones)
                return 0

            lax.fori_loop(0, EQ // 160, chunk, 0)
            pltpu.sync_copy(
                hist, degp_ref.at[pl.ds(p * (C * 2 * N) + (cut * 2 + which) * N, N)]
            )


@functools.partial(
    pl.kernel,
    mesh=_mesh,
    out_type=jax.ShapeDtypeStruct((C, 2, N, HD), jnp.float32),
    scratch_types=[
        pltpu.VMEM_SHARED((N + 48, HD), jnp.float32),
        pltpu.VMEM((K, HD), jnp.float32),
        pltpu.VMEM((K, HD), jnp.float32),
        pltpu.VMEM((NCH // 2, K), jnp.int32),
        pltpu.VMEM((NCH // 2, K), jnp.int32),
        pltpu.VMEM((48, HD), jnp.float32),
        pltpu.SemaphoreType.DMA,
        pltpu.SemaphoreType.DMA,
    ],
)
def _agg_sc(hn_ref, srcs3_ref, dsts3_ref, out_ref, acc, rows0, rows1,
            idx_s2, idx_d2, zbuf, sem0, sem1):
    c = lax.axis_index("c")
    s = lax.axis_index("s")

    def zb(i, _):
        for k in range(HD // 16):
            zbuf[i, pl.ds(k * 16, 16)] = jnp.zeros((16,), jnp.float32)
        return 0

    lax.fori_loop(0, 48, zb, 0)
    NH = NCH // 2

    for hl in range(2):

        @pl.when(c == hl)
        def _():
            for cut in range(C):
                table = hn_ref.at[cut, hl]
                base = s * 624
                for z in range(14):
                    pltpu.sync_copy(zbuf, acc.at[pl.ds(base + z * 48, 48)])
                plsc.subcore_barrier()

                for h2 in range(2):
                    pltpu.sync_copy(
                        srcs3_ref.at[cut * NT + s, pl.ds(h2 * NH, NH)], idx_s2)
                    pltpu.sync_copy(
                        dsts3_ref.at[cut * NT + s, pl.ds(h2 * NH, NH)], idx_d2)
                    pltpu.async_copy(table.at[idx_s2.at[0]], rows0, sem0)

                    def body(k, _):
                        i0 = 2 * k
                        pltpu.async_copy(table.at[idx_s2.at[i0 + 1]], rows1, sem1)
                        pltpu.make_async_copy(
                            table.at[idx_s2.at[i0]], rows0, sem0).wait()
                        pltpu.sync_copy(rows0, acc.at[idx_d2.at[i0]], add=True)
                        pltpu.async_copy(table.at[idx_s2.at[i0 + 2]], rows0, sem0)
                        pltpu.make_async_copy(
                            table.at[idx_s2.at[i0 + 1]], rows1, sem1).wait()
                        pltpu.sync_copy(rows1, acc.at[idx_d2.at[i0 + 1]], add=True)
                        return 0

                    lax.fori_loop(0, NH // 2 - 1, body, 0)
                    pltpu.async_copy(table.at[idx_s2.at[NH - 1]], rows1, sem1)
                    pltpu.make_async_copy(
                        table.at[idx_s2.at[NH - 2]], rows0, sem0).wait()
                    pltpu.sync_copy(rows0, acc.at[idx_d2.at[NH - 2]], add=True)
                    pltpu.make_async_copy(
                        table.at[idx_s2.at[NH - 1]], rows1, sem1).wait()
                    pltpu.sync_copy(rows1, acc.at[idx_d2.at[NH - 1]], add=True)

                plsc.subcore_barrier()
                pltpu.sync_copy(
                    acc.at[pl.ds(base, 640)], out_ref.at[cut, hl, pl.ds(base, 640)]
                )
                plsc.subcore_barrier()


_BN = 1000


def _scale_body(x_ref, degt_ref, out_ref):
    deg = jnp.sum(degt_ref[0], axis=-1)
    ns = lax.rsqrt(jnp.maximum(deg[:, 0:1], 1.0))
    h = x_ref[0] * ns
    out_ref[0, 0] = h[:, :HD]
    out_ref[0, 1] = h[:, HD:]


def _scale_tc(x, degt):
    return pl.pallas_call(
        _scale_body,
        grid=(C, N // _BN),
        in_specs=[
            pl.BlockSpec((1, _BN, D), lambda ci, ni: (ci, ni, 0)),
            pl.BlockSpec((1, _BN, 2, 4), lambda ci, ni: (ci, ni, 0, 0)),
        ],
        out_specs=pl.BlockSpec((1, 2, _BN, HD), lambda ci, ni: (ci, 0, ni, 0)),
        out_shape=jax.ShapeDtypeStruct((C, 2, N, HD), jnp.float32),
    )(x, degt)


def _layer_body(mid, agg_ref, degt_ref, w_ref, b_ref, out_ref):
    a = jnp.concatenate([agg_ref[0, 0], agg_ref[0, 1]], axis=-1)
    deg = jnp.sum(degt_ref[0], axis=-1)
    nd = lax.rsqrt(jnp.maximum(deg[:, 1:2], 1.0))
    a = a * nd
    h = (
        jnp.dot(a, w_ref[0], preferred_element_type=jnp.float32,
                precision=lax.Precision.HIGHEST)
        + b_ref[0, 0]
    )
    h = jnp.where(h >= 0.0, h, 0.01 * h)
    if mid:
        ns = lax.rsqrt(jnp.maximum(deg[:, 0:1], 1.0))
        h = h * ns
        out_ref[0, 0] = h[:, :HD]
        out_ref[0, 1] = h[:, HD:]
    else:
        out_ref[0] = h


def _layer_tc(agg, degt, w, b, mid):
    if mid:
        out_spec = pl.BlockSpec((1, 2, _BN, HD), lambda ci, ni: (ci, 0, ni, 0))
        out_shape = jax.ShapeDtypeStruct((C, 2, N, HD), jnp.float32)
    else:
        out_spec = pl.BlockSpec((1, _BN, D), lambda ci, ni: (ci, ni, 0))
        out_shape = jax.ShapeDtypeStruct((C, N, D), jnp.float32)
    return pl.pallas_call(
        functools.partial(_layer_body, mid),
        grid=(C, N // _BN),
        in_specs=[
            pl.BlockSpec((1, 2, _BN, HD), lambda ci, ni: (ci, 0, ni, 0)),
            pl.BlockSpec((1, _BN, 2, 4), lambda ci, ni: (ci, ni, 0, 0)),
            pl.BlockSpec((1, D, D), lambda ci, ni: (ci, 0, 0)),
            pl.BlockSpec((1, 1, D), lambda ci, ni: (ci, 0, 0)),
        ],
        out_specs=out_spec,
        out_shape=out_shape,
    )(agg, degt, w, b.reshape(C, 1, D))


def _gru_body(feats_ref, wi_ref, wh_ref, bi_ref, bh_ref, out_ref):
    h = jnp.zeros((_BN, D), jnp.float32)
    wi = wi_ref[...]
    wh = wh_ref[...]
    bi = bi_ref[0]
    bh = bh_ref[0]
    dn = (((1,), (1,)), ((), ()))
    for t in range(C):
        xt = feats_ref[t]
        gi = lax.dot_general(xt, wi, dn, preferred_element_type=jnp.float32,
                             precision=lax.Precision.HIGHEST) + bi
        gh = lax.dot_general(h, wh, dn, preferred_element_type=jnp.float32,
                             precision=lax.Precision.HIGHEST) + bh
        r = jax.nn.sigmoid(gi[:, :D] + gh[:, :D])
        z = jax.nn.sigmoid(gi[:, D : 2 * D] + gh[:, D : 2 * D])
        n = jnp.tanh(gi[:, 2 * D :] + r * gh[:, 2 * D :])
        h = (1.0 - z) * n + z * h
    out_ref[...] = h


def _gru_tc(feats, w_ih, w_hh, b_ih, b_hh):
    return pl.pallas_call(
        _gru_body,
        grid=(N // _BN,),
        in_specs=[
            pl.BlockSpec((C, _BN, D), lambda ni: (0, ni, 0)),
            pl.BlockSpec((3 * D, D), lambda ni: (0, 0)),
            pl.BlockSpec((3 * D, D), lambda ni: (0, 0)),
            pl.BlockSpec((1, 3 * D), lambda ni: (0, 0)),
            pl.BlockSpec((1, 3 * D), lambda ni: (0, 0)),
        ],
        out_specs=pl.BlockSpec((_BN, D), lambda ni: (ni, 0)),
        out_shape=jax.ShapeDtypeStruct((N, D), jnp.float32),
    )(feats, w_ih, w_hh, b_ih, b_hh)


def kernel(x, edge_index, W_gnn, b_gnn, W_ih, W_hh, b_ih, b_hh):
    edges = edge_index.astype(jnp.int32)
    srcs = edges[:, 0, :].reshape(C * E)
    dsts = edges[:, 1, :].reshape(C * E)
    srcs_t = srcs.reshape(C * NT, EPT)
    dsts_t = dsts.reshape(C * NT, EPT)
    pad_s = jnp.zeros((C * NT, PAD), jnp.int32)
    pad_d = jnp.broadcast_to(
        N + (jnp.arange(PAD, dtype=jnp.int32) % 16), (C * NT, PAD))
    srcs3 = jnp.concatenate([srcs_t, pad_s], 1).reshape(C * NT, NCH, K)
    dsts3 = jnp.concatenate([dsts_t, pad_d], 1).reshape(C * NT, NCH, K)
    srcs4 = srcs.reshape(C * 4, E // 4)
    dsts4 = dsts.reshape(C * 4, E // 4)
    degp = _deg_sc(srcs4, dsts4)
    degt = degp.reshape(4, C, 2, N).transpose(1, 3, 2, 0)
    hn0 = _scale_tc(x, degt)
    agg0 = _agg_sc(hn0, srcs3, dsts3)
    hn1 = _layer_tc(agg0, degt, W_gnn[:, 0], b_gnn[:, 0], mid=True)
    agg1 = _agg_sc(hn1, srcs3, dsts3)
    feats = _layer_tc(agg1, degt, W_gnn[:, 1], b_gnn[:, 1], mid=False)
    return _gru_tc(feats, W_ih, W_hh, b_ih.reshape(1, 3 * D),
                   b_hh.reshape(1, 3 * D))


# speedup vs baseline: 2.8159x; 1.0541x over previous
import functools

import jax
import jax.numpy as jnp
from jax import lax
from jax.experimental import pallas as pl
from jax.experimental.pallas import tpu as pltpu
from jax.experimental.pallas import tpu_sc as plsc

C = 4
N = 10000
E = 160000
D = 256
HD = 128
NT = 16
EPT = E // NT
K = 128
EPTP = 10240
NCH = EPTP // K
PAD = EPTP - EPT
NPT = N // NT

_mesh = plsc.VectorSubcoreMesh(core_axis_name="c", subcore_axis_name="s")


@functools.partial(
    pl.kernel,
    mesh=_mesh,
    compiler_params=pltpu.CompilerParams(needs_layout_passes=False),
    out_type=jax.ShapeDtypeStruct((4 * C * 2 * N,), jnp.float32),
    scratch_types=[
        pltpu.VMEM((N,), jnp.float32),
        pltpu.VMEM((E // 4,), jnp.int32),
    ],
)
def _deg_sc(srcs4_ref, dsts4_ref, degp_ref, hist, idxb):
    c = lax.axis_index("c")
    s = lax.axis_index("s")
    j = c * 4 + s // 4
    p = s % 4
    EQ = E // 4
    for jid in range(8):

        @pl.when(j == jid)
        def _():
            cut, which = jid // 2, jid % 2
            e_ref = srcs4_ref if which == 0 else dsts4_ref

            def zb(i, _):
                hist[pl.ds(i * 16, 16)] = jnp.zeros((16,), jnp.float32)
                return 0

            lax.fori_loop(0, N // 16, zb, 0)
            pltpu.sync_copy(e_ref.at[cut * 4 + p], idxb)
            ones = jnp.ones((16,), jnp.float32)

            def chunk(i, _):
                for k in range(10):
                    v = idxb[pl.ds(i * 160 + k * 16, 16)]
                    plsc.addupdate_scatter(hist, [v], ones)
                return 0

            lax.fori_loop(0, EQ // 160, chunk, 0)
            pltpu.sync_copy(
                hist, degp_ref.at[pl.ds(p * (C * 2 * N) + (cut * 2 + which) * N, N)]
            )


@functools.partial(
    pl.kernel,
    mesh=_mesh,
    out_type=jax.ShapeDtypeStruct((2, N, HD), jnp.float32),
    scratch_types=[
        pltpu.VMEM_SHARED((N + 48, HD), jnp.float32),
        pltpu.VMEM((K, HD), jnp.float32),
        pltpu.VMEM((K, HD), jnp.float32),
        pltpu.VMEM((NCH // 2, K), jnp.int32),
        pltpu.VMEM((NCH // 2, K), jnp.int32),
        pltpu.VMEM((48, HD), jnp.float32),
        pltpu.SemaphoreType.DMA,
        pltpu.SemaphoreType.DMA,
        pltpu.SemaphoreType.DMA,
        pltpu.SemaphoreType.DMA,
    ],
)
def _agg_sc(hn_ref, srcs3_ref, dsts3_ref, out_ref, acc, rows0, rows1,
            idx_s2, idx_d2, zbuf, sem0, sem1, sem2, sem3):
    c = lax.axis_index("c")
    s = lax.axis_index("s")

    def zb(i, _):
        for k in range(HD // 16):
            zbuf[i, pl.ds(k * 16, 16)] = jnp.zeros((16,), jnp.float32)
        return 0

    lax.fori_loop(0, 48, zb, 0)
    NH = NCH // 2

    for hl in range(2):

        @pl.when(c == hl)
        def _():
            table = hn_ref.at[hl]
            base = s * 624
            for z in range(14):
                pltpu.sync_copy(zbuf, acc.at[pl.ds(base + z * 48, 48)])
            plsc.subcore_barrier()

            for h2 in range(2):
                pltpu.sync_copy(srcs3_ref.at[s, pl.ds(h2 * NH, NH)], idx_s2)
                pltpu.sync_copy(dsts3_ref.at[s, pl.ds(h2 * NH, NH)], idx_d2)
                pltpu.async_copy(table.at[idx_s2.at[0]], rows0, sem0)
                pltpu.async_copy(table.at[idx_s2.at[1]], rows1, sem1)

                def body(k, _):
                    i0 = 2 * k
                    pltpu.make_async_copy(
                        table.at[idx_s2.at[i0]], rows0, sem0).wait()
                    pltpu.sync_copy(rows0, acc.at[idx_d2.at[i0]], add=True)
                    pltpu.async_copy(table.at[idx_s2.at[i0 + 2]], rows0, sem0)
                    pltpu.make_async_copy(
                        table.at[idx_s2.at[i0 + 1]], rows1, sem1).wait()
                    pltpu.sync_copy(rows1, acc.at[idx_d2.at[i0 + 1]], add=True)
                    pltpu.async_copy(table.at[idx_s2.at[i0 + 3]], rows1, sem1)
                    return 0

                lax.fori_loop(0, NH // 2 - 1, body, 0)
                i0 = NH - 2
                pltpu.make_async_copy(
                    table.at[idx_s2.at[i0]], rows0, sem0).wait()
                pltpu.sync_copy(rows0, acc.at[idx_d2.at[i0]], add=True)
                pltpu.make_async_copy(
                    table.at[idx_s2.at[i0 + 1]], rows1, sem1).wait()
                pltpu.sync_copy(rows1, acc.at[idx_d2.at[i0 + 1]], add=True)

            plsc.subcore_barrier()
            pltpu.sync_copy(
                acc.at[pl.ds(base, 640)], out_ref.at[hl, pl.ds(base, 640)]
            )


_BN = 1000


def _scale_body(x_ref, degt_ref, out_ref):
    deg = jnp.sum(degt_ref[...], axis=-1)
    ns = lax.rsqrt(jnp.maximum(deg[:, 0:1], 1.0))
    h = x_ref[...] * ns
    out_ref[0] = h[:, :HD]
    out_ref[1] = h[:, HD:]


def _scale_tc(x, degt):
    return pl.pallas_call(
        _scale_body,
        grid=(N // _BN,),
        in_specs=[
            pl.BlockSpec((_BN, D), lambda ni: (ni, 0)),
            pl.BlockSpec((_BN, 2, 4), lambda ni: (ni, 0, 0)),
        ],
        out_specs=pl.BlockSpec((2, _BN, HD), lambda ni: (0, ni, 0)),
        out_shape=jax.ShapeDtypeStruct((2, N, HD), jnp.float32),
    )(x, degt)


def _layer_mid_body(agg_ref, degt_ref, w_ref, b_ref, out_ref):
    a = jnp.concatenate([agg_ref[0], agg_ref[1]], axis=-1)
    deg = jnp.sum(degt_ref[...], axis=-1)
    nd = lax.rsqrt(jnp.maximum(deg[:, 1:2], 1.0))
    a = a * nd
    h = (
        jnp.dot(a, w_ref[...], preferred_element_type=jnp.float32,
                precision=lax.Precision.HIGHEST)
        + b_ref[0]
    )
    h = jnp.where(h >= 0.0, h, 0.01 * h)
    ns = lax.rsqrt(jnp.maximum(deg[:, 0:1], 1.0))
    h = h * ns
    out_ref[0] = h[:, :HD]
    out_ref[1] = h[:, HD:]


def _layer_mid_tc(agg, degt, w, b):
    return pl.pallas_call(
        _layer_mid_body,
        grid=(N // _BN,),
        in_specs=[
            pl.BlockSpec((2, _BN, HD), lambda ni: (0, ni, 0)),
            pl.BlockSpec((_BN, 2, 4), lambda ni: (ni, 0, 0)),
            pl.BlockSpec((D, D), lambda ni: (0, 0)),
            pl.BlockSpec((1, D), lambda ni: (0, 0)),
        ],
        out_specs=pl.BlockSpec((2, _BN, HD), lambda ni: (0, ni, 0)),
        out_shape=jax.ShapeDtypeStruct((2, N, HD), jnp.float32),
    )(agg, degt, w, b.reshape(1, D))


def _layer_last_body(agg_ref, degt_ref, w_ref, b_ref, wi_ref, bi_ref, gi_ref):
    a = jnp.concatenate([agg_ref[0], agg_ref[1]], axis=-1)
    deg = jnp.sum(degt_ref[...], axis=-1)
    nd = lax.rsqrt(jnp.maximum(deg[:, 1:2], 1.0))
    a = a * nd
    h = (
        jnp.dot(a, w_ref[...], preferred_element_type=jnp.float32,
                precision=lax.Precision.HIGHEST)
        + b_ref[0]
    )
    h = jnp.where(h >= 0.0, h, 0.01 * h)
    dn = (((1,), (1,)), ((), ()))
    gi_ref[...] = lax.dot_general(
        h, wi_ref[...], dn, preferred_element_type=jnp.float32,
        precision=lax.Precision.HIGHEST) + bi_ref[0]


def _layer_last_tc(agg, degt, w, b, w_ih, b_ih):
    return pl.pallas_call(
        _layer_last_body,
        grid=(N // _BN,),
        in_specs=[
            pl.BlockSpec((2, _BN, HD), lambda ni: (0, ni, 0)),
            pl.BlockSpec((_BN, 2, 4), lambda ni: (ni, 0, 0)),
            pl.BlockSpec((D, D), lambda ni: (0, 0)),
            pl.BlockSpec((1, D), lambda ni: (0, 0)),
            pl.BlockSpec((3 * D, D), lambda ni: (0, 0)),
            pl.BlockSpec((1, 3 * D), lambda ni: (0, 0)),
        ],
        out_specs=pl.BlockSpec((_BN, 3 * D), lambda ni: (ni, 0)),
        out_shape=jax.ShapeDtypeStruct((N, 3 * D), jnp.float32),
    )(agg, degt, w, b.reshape(1, D), w_ih, b_ih)


def _gru_body(g0_ref, g1_ref, g2_ref, g3_ref, wh_ref, bh_ref, out_ref):
    gi_refs = (g0_ref, g1_ref, g2_ref, g3_ref)
    h = jnp.zeros((_BN, D), jnp.float32)
    wh = wh_ref[...]
    bh = bh_ref[0]
    dn = (((1,), (1,)), ((), ()))
    for t in range(C):
        gi = gi_refs[t][...]
        gh = lax.dot_general(h, wh, dn, preferred_element_type=jnp.float32,
                             precision=lax.Precision.HIGHEST) + bh
        r = jax.nn.sigmoid(gi[:, :D] + gh[:, :D])
        z = jax.nn.sigmoid(gi[:, D : 2 * D] + gh[:, D : 2 * D])
        n = jnp.tanh(gi[:, 2 * D :] + r * gh[:, 2 * D :])
        h = (1.0 - z) * n + z * h
    out_ref[...] = h


def _gru_tc(gis, w_hh, b_hh):
    return pl.pallas_call(
        _gru_body,
        grid=(N // _BN,),
        in_specs=[
            pl.BlockSpec((_BN, 3 * D), lambda ni: (ni, 0)),
            pl.BlockSpec((_BN, 3 * D), lambda ni: (ni, 0)),
            pl.BlockSpec((_BN, 3 * D), lambda ni: (ni, 0)),
            pl.BlockSpec((_BN, 3 * D), lambda ni: (ni, 0)),
            pl.BlockSpec((3 * D, D), lambda ni: (0, 0)),
            pl.BlockSpec((1, 3 * D), lambda ni: (0, 0)),
        ],
        out_specs=pl.BlockSpec((_BN, D), lambda ni: (ni, 0)),
        out_shape=jax.ShapeDtypeStruct((N, D), jnp.float32),
    )(*gis, w_hh, b_hh)


def kernel(x, edge_index, W_gnn, b_gnn, W_ih, W_hh, b_ih, b_hh):
    edges = edge_index.astype(jnp.int32)
    srcs = edges[:, 0, :].reshape(C * E)
    dsts = edges[:, 1, :].reshape(C * E)
    srcs_t = srcs.reshape(C * NT, EPT)
    dsts_t = dsts.reshape(C * NT, EPT)
    pad_s = jnp.zeros((C * NT, PAD), jnp.int32)
    pad_d = jnp.broadcast_to(
        N + (jnp.arange(PAD, dtype=jnp.int32) % 16), (C * NT, PAD))
    srcs3 = jnp.concatenate([srcs_t, pad_s], 1).reshape(C * NT, NCH, K)
    dsts3 = jnp.concatenate([dsts_t, pad_d], 1).reshape(C * NT, NCH, K)
    srcs4 = srcs.reshape(C * 4, E // 4)
    dsts4 = dsts.reshape(C * 4, E // 4)
    degp = _deg_sc(srcs4, dsts4)
    degt = degp.reshape(4, C, 2, N).transpose(1, 3, 2, 0)
    src3c = srcs3.reshape(C, NT, NCH, K)
    dst3c = dsts3.reshape(C, NT, NCH, K)
    gis = []
    bi2 = b_ih.reshape(1, 3 * D)
    for cc in range(C):
        hn0 = _scale_tc(x[cc], degt[cc])
        agg0 = _agg_sc(hn0, src3c[cc], dst3c[cc])
        hn1 = _layer_mid_tc(agg0, degt[cc], W_gnn[cc, 0], b_gnn[cc, 0])
        agg1 = _agg_sc(hn1, src3c[cc], dst3c[cc])
        gis.append(
            _layer_last_tc(agg1, degt[cc], W_gnn[cc, 1], b_gnn[cc, 1],
                           W_ih, bi2))
    return _gru_tc(gis, W_hh, b_hh.reshape(1, 3 * D))
